# baseline (speedup 1.0000x reference)
import functools
import math

import numpy as np
import jax
import jax.numpy as jnp
from jax import lax
from jax.experimental import pallas as pl
from jax.experimental.pallas import tpu as pltpu
from jax.experimental.pallas import tpu_sc as plsc

PS = 16
MASK_RATIO = 0.75
MIN_MASK = 4
MAX_MASK = 48

B, C, H, W = 4, 96, 384, 384
NPH, NPW = H // PS, W // PS
TOTAL = NPH * NPW

_num_masked = max(MIN_MASK, min(int(TOTAL * MASK_RATIO), MAX_MASK))
_bs = int(math.sqrt(_num_masked))
_ch, _cw = NPH // 2, NPW // 2
_MASK_IDS = [i * NPW + j
             for i in range(max(0, _ch - _bs // 2), min(NPH, _ch + _bs // 2))
             for j in range(max(0, _cw - _bs // 2), min(NPW, _cw + _bs // 2))]
_mask_row = np.zeros(TOTAL, dtype=bool)
_mask_row[_MASK_IDS] = True
_KEEP = np.nonzero(~_mask_row)[0]
NKEEP = len(_KEEP)

_mi = np.asarray(_MASK_IDS) // NPW
_mj = np.asarray(_MASK_IDS) % NPW
R0, R1 = int(_mi.min()) * PS, (int(_mi.max()) + 1) * PS
C0, C1 = int(_mj.min()) * PS, (int(_mj.max()) + 1) * PS

_MASK_CONST = np.tile(_mask_row[None, :], (B, 1))

KPAD = 544
NROWS_OUT = B * KPAD * C * PS
NROWS_TAB = B * C * H * NPW
NW = 32
RPW = NROWS_OUT // NW
DMAS_PER_STEP = 24
ROWS_PER_STEP = DMAS_PER_STEP * 128
STEPS = RPW // ROWS_PER_STEP
assert RPW % ROWS_PER_STEP == 0


def _gather_index() -> np.ndarray:
    b = np.arange(B)[:, None, None, None]
    k = np.arange(KPAD)[None, :, None, None]
    c = np.arange(C)[None, None, :, None]
    pi = np.arange(PS)[None, None, None, :]
    p = _KEEP[np.minimum(k, NKEEP - 1)]
    i = p // NPW
    j = p % NPW
    tr = 2 * i + pi // 8
    r = ((((b * C + c) * 48 + tr) * 3 + j // 8) * 8 + pi % 8) * 8 + j % 8
    r = np.where(k > NKEEP - 1, 0, r).astype(np.int32)
    return r.reshape(NW, STEPS, DMAS_PER_STEP, 128)


_IDX4 = _gather_index()


def _sc_gather(x_rows, idx4):
    mesh = plsc.VectorSubcoreMesh(core_axis_name="c", subcore_axis_name="s")

    @functools.partial(
        pl.kernel,
        mesh=mesh,
        compiler_params=pltpu.CompilerParams(use_tc_tiling_on_sc=False),
        out_type=jax.ShapeDtypeStruct((NROWS_OUT, PS), jnp.float32),
        scratch_types=[
            pltpu.VMEM((DMAS_PER_STEP, 128), jnp.int32),
            pltpu.VMEM((DMAS_PER_STEP, 128), jnp.int32),
            pltpu.VMEM((ROWS_PER_STEP, PS), jnp.float32),
            pltpu.VMEM((ROWS_PER_STEP, PS), jnp.float32),
            pltpu.SemaphoreType.DMA,
            pltpu.SemaphoreType.DMA,
        ],
    )
    def k(x_hbm, idx_hbm, out_hbm, idx_a, idx_b, buf_a, buf_b, sem_g,
          sem_w):
        wid = lax.axis_index("c") * 16 + lax.axis_index("s")
        base = wid * RPW

        def fire(idx_v, buf):
            return [
                pltpu.async_copy(x_hbm.at[idx_v.at[d]],
                                 buf.at[pl.ds(d * 128, 128)], sem_g)
                for d in range(DMAS_PER_STEP)
            ]

        def drain_gathers(buf):
            pltpu.make_async_copy(
                x_hbm.at[pl.ds(0, ROWS_PER_STEP)], buf, sem_g).wait()

        def drain_write(buf):
            pltpu.make_async_copy(
                buf, out_hbm.at[pl.ds(0, ROWS_PER_STEP)], sem_w).wait()

        def out_at(t):
            return out_hbm.at[pl.ds(base + t * ROWS_PER_STEP,
                                    ROWS_PER_STEP)]

        pltpu.sync_copy(idx_hbm.at[wid, 0], idx_a)
        fire(idx_a, buf_a)

        def pair(t, carry):
            a, bmo = 2 * t, 2 * t + 1
            pltpu.sync_copy(idx_hbm.at[wid, bmo], idx_b)
            drain_gathers(buf_a)

            @pl.when(t > 0)
            def _():
                drain_write(buf_b)

            fire(idx_b, buf_b)
            pltpu.async_copy(buf_a, out_at(a), sem_w)

            @pl.when(t < STEPS // 2 - 1)
            def _():
                pltpu.sync_copy(idx_hbm.at[wid, 2 * t + 2], idx_a)

            drain_gathers(buf_b)
            drain_write(buf_a)

            @pl.when(t < STEPS // 2 - 1)
            def _():
                fire(idx_a, buf_a)

            pltpu.async_copy(buf_b, out_at(bmo), sem_w)
            return carry

        lax.fori_loop(0, STEPS // 2, pair, 0)
        drain_write(buf_b)

    return k(x_rows, idx4)


_MROWS = 9216


def _tc_masked_copy(x2):
    def body(in_ref, out_ref):
        g = pl.program_id(0)
        h = (lax.broadcasted_iota(jnp.int32, (_MROWS, W), 0) + g * _MROWS) % H
        c = lax.broadcasted_iota(jnp.int32, (_MROWS, W), 1)
        inside = (h >= R0) & (h < R1) & (c >= C0) & (c < C1)
        out_ref[...] = jnp.where(inside, 0.0, in_ref[...])

    return pl.pallas_call(
        body,
        grid=(B * C * H // _MROWS,),
        in_specs=[pl.BlockSpec((_MROWS, W), lambda g: (g, 0))],
        out_specs=pl.BlockSpec((_MROWS, W), lambda g: (g, 0)),
        out_shape=jax.ShapeDtypeStruct((B * C * H, W), jnp.float32),
    )(x2)


def kernel(x):
    x_rows = (x.reshape(B, C, 48, 8, 3, 128)
              .transpose(0, 1, 2, 4, 3, 5)
              .reshape(NROWS_TAB, PS))
    out2d = _sc_gather(x_rows, jnp.asarray(_IDX4))

    x2 = x.reshape(B * C * H, W)
    masked_input = _tc_masked_copy(x2).reshape(B, C, H, W)

    unmasked_patches = out2d.reshape(B, KPAD, C * PS * PS)[:, :NKEEP]

    mask = jnp.asarray(_MASK_CONST)
    return (masked_input, mask, unmasked_patches)


# speedup vs baseline: 1.0730x; 1.0730x over previous
import functools
import math

import numpy as np
import jax
import jax.numpy as jnp
from jax import lax
from jax.experimental import pallas as pl
from jax.experimental.pallas import tpu as pltpu
from jax.experimental.pallas import tpu_sc as plsc

PS = 16
MASK_RATIO = 0.75
MIN_MASK = 4
MAX_MASK = 48

B, C, H, W = 4, 96, 384, 384
NPH, NPW = H // PS, W // PS
TOTAL = NPH * NPW

_num_masked = max(MIN_MASK, min(int(TOTAL * MASK_RATIO), MAX_MASK))
_bs = int(math.sqrt(_num_masked))
_ch, _cw = NPH // 2, NPW // 2
_MASK_IDS = [i * NPW + j
             for i in range(max(0, _ch - _bs // 2), min(NPH, _ch + _bs // 2))
             for j in range(max(0, _cw - _bs // 2), min(NPW, _cw + _bs // 2))]
_mask_row = np.zeros(TOTAL, dtype=bool)
_mask_row[_MASK_IDS] = True
_KEEP = np.nonzero(~_mask_row)[0]
NKEEP = len(_KEEP)

_mi = np.asarray(_MASK_IDS) // NPW
_mj = np.asarray(_MASK_IDS) % NPW
R0, R1 = int(_mi.min()) * PS, (int(_mi.max()) + 1) * PS
C0, C1 = int(_mj.min()) * PS, (int(_mj.max()) + 1) * PS

_MASK_CONST = np.tile(_mask_row[None, :], (B, 1))

KPAD = 544
NROWS_OUT = B * KPAD * C * PS
NROWS_TAB = B * C * H * NPW
NW = 32
RPW = NROWS_OUT // NW
DMAS_PER_STEP = 12
ROWS_PER_STEP = DMAS_PER_STEP * 128
STEPS = RPW // ROWS_PER_STEP
assert RPW % ROWS_PER_STEP == 0


def _gather_index() -> np.ndarray:
    b = np.arange(B)[:, None, None, None]
    k = np.arange(KPAD)[None, :, None, None]
    c = np.arange(C)[None, None, :, None]
    pi = np.arange(PS)[None, None, None, :]
    p = _KEEP[np.minimum(k, NKEEP - 1)]
    i = p // NPW
    j = p % NPW
    tr = 2 * i + pi // 8
    r = ((((b * C + c) * 48 + tr) * 3 + j // 8) * 8 + pi % 8) * 8 + j % 8
    r = np.where(k > NKEEP - 1, 0, r).astype(np.int32)
    return r.reshape(NW, STEPS, DMAS_PER_STEP, 128)


_IDX4 = _gather_index()


def _sc_gather(x_rows, idx4):
    mesh = plsc.VectorSubcoreMesh(core_axis_name="c", subcore_axis_name="s")

    @functools.partial(
        pl.kernel,
        mesh=mesh,
        compiler_params=pltpu.CompilerParams(use_tc_tiling_on_sc=False),
        out_type=jax.ShapeDtypeStruct((NROWS_OUT, PS), jnp.float32),
        scratch_types=[
            pltpu.VMEM((DMAS_PER_STEP, 128), jnp.int32),
            pltpu.VMEM((DMAS_PER_STEP, 128), jnp.int32),
            pltpu.VMEM((ROWS_PER_STEP, PS), jnp.float32),
            pltpu.VMEM((ROWS_PER_STEP, PS), jnp.float32),
            pltpu.SemaphoreType.DMA,
            pltpu.SemaphoreType.DMA,
        ],
    )
    def k(x_hbm, idx_hbm, out_hbm, idx_a, idx_b, buf_a, buf_b, sem_g,
          sem_w):
        wid = lax.axis_index("c") * 16 + lax.axis_index("s")
        base = wid * RPW

        def fire(idx_v, buf):
            return [
                pltpu.async_copy(x_hbm.at[idx_v.at[d]],
                                 buf.at[pl.ds(d * 128, 128)], sem_g)
                for d in range(DMAS_PER_STEP)
            ]

        def drain_gathers(buf):
            pltpu.make_async_copy(
                x_hbm.at[pl.ds(0, ROWS_PER_STEP)], buf, sem_g).wait()

        def drain_write(buf):
            pltpu.make_async_copy(
                buf, out_hbm.at[pl.ds(0, ROWS_PER_STEP)], sem_w).wait()

        def out_at(t):
            return out_hbm.at[pl.ds(base + t * ROWS_PER_STEP,
                                    ROWS_PER_STEP)]

        pltpu.sync_copy(idx_hbm.at[wid, 0], idx_a)
        fire(idx_a, buf_a)

        def pair(t, carry):
            a, bmo = 2 * t, 2 * t + 1
            pltpu.sync_copy(idx_hbm.at[wid, bmo], idx_b)
            drain_gathers(buf_a)

            @pl.when(t > 0)
            def _():
                drain_write(buf_b)

            fire(idx_b, buf_b)
            pltpu.async_copy(buf_a, out_at(a), sem_w)

            @pl.when(t < STEPS // 2 - 1)
            def _():
                pltpu.sync_copy(idx_hbm.at[wid, 2 * t + 2], idx_a)

            drain_gathers(buf_b)
            drain_write(buf_a)

            @pl.when(t < STEPS // 2 - 1)
            def _():
                fire(idx_a, buf_a)

            pltpu.async_copy(buf_b, out_at(bmo), sem_w)
            return carry

        lax.fori_loop(0, STEPS // 2, pair, 0)
        drain_write(buf_b)

    return k(x_rows, idx4)


_MROWS = 9216


def _tc_masked_copy(x2):
    def body(in_ref, out_ref):
        g = pl.program_id(0)
        h = (lax.broadcasted_iota(jnp.int32, (_MROWS, W), 0) + g * _MROWS) % H
        c = lax.broadcasted_iota(jnp.int32, (_MROWS, W), 1)
        inside = (h >= R0) & (h < R1) & (c >= C0) & (c < C1)
        out_ref[...] = jnp.where(inside, 0.0, in_ref[...])

    return pl.pallas_call(
        body,
        grid=(B * C * H // _MROWS,),
        in_specs=[pl.BlockSpec((_MROWS, W), lambda g: (g, 0))],
        out_specs=pl.BlockSpec((_MROWS, W), lambda g: (g, 0)),
        out_shape=jax.ShapeDtypeStruct((B * C * H, W), jnp.float32),
    )(x2)


def kernel(x):
    x_rows = (x.reshape(B, C, 48, 8, 3, 128)
              .transpose(0, 1, 2, 4, 3, 5)
              .reshape(NROWS_TAB, PS))
    out2d = _sc_gather(x_rows, jnp.asarray(_IDX4))

    x2 = x.reshape(B * C * H, W)
    masked_input = _tc_masked_copy(x2).reshape(B, C, H, W)

    unmasked_patches = out2d.reshape(B, KPAD, C * PS * PS)[:, :NKEEP]

    mask = jnp.asarray(_MASK_CONST)
    return (masked_input, mask, unmasked_patches)
